# Initial kernel scaffold; baseline (speedup 1.0000x reference)
import jax
import jax.numpy as jnp
from jax.experimental import pallas as pl


def kernel(inputs, hx, rows, cols, vals, W_ru, b_ru, W_c, b_c):
    raise NotImplementedError("write your pallas kernel here")


# speedup vs baseline: 1.8838x; 1.8838x over previous
import functools

import jax
import jax.numpy as jnp
from jax import lax
from jax.experimental import pallas as pl
from jax.experimental.pallas import tpu as pltpu
from jax.experimental.pallas import tpu_sc as plsc

N_NODES = 10000
N_PAD = 10240
BATCH = 8
IN_DIM = 2
UNITS = 64
CHEB_K = 2
NUM_EDGES = 160000

FEAT = IN_DIM + UNITS
FEAT_PAD = 72
ROW_W = BATCH * FEAT_PAD
NCHUNK = 6
CW = ROW_W // NCHUNK
NSUB = 16
NCORE = 2
EPT = NUM_EDGES // NSUB
BE = 128
NBLK = EPT // BE
TAIL = EPT - NBLK * BE
RPT = N_PAD // NSUB
RB = 128
NRB = RPT // RB
NLANE = 16
CWV = CW // NLANE


def _make_spmm(cheb: bool):
    mesh = plsc.VectorSubcoreMesh(
        core_axis_name="c", subcore_axis_name="s",
        num_cores=NCORE, num_subcores=NSUB)

    scratch = [
        pltpu.VMEM((BE,), jnp.int32),
        pltpu.VMEM((BE,), jnp.int32),
        pltpu.VMEM((BE,), jnp.float32),
        pltpu.VMEM((BE, CW), jnp.float32),
        pltpu.VMEM((TAIL,), jnp.int32),
        pltpu.VMEM((TAIL,), jnp.int32),
        pltpu.VMEM((TAIL,), jnp.float32),
        pltpu.VMEM((TAIL, CW), jnp.float32),
        pltpu.VMEM((RB, CW), jnp.float32),
        pltpu.VMEM((RB, CW), jnp.float32),
        pltpu.VMEM((RB, CW), jnp.float32),
        pltpu.VMEM_SHARED((N_PAD, CW), jnp.float32),
        pltpu.SemaphoreType.DMA,
    ]

    def body(*refs):
        if cheb:
            (x_hbm, xp_hbm, gi_hbm, si_hbm, vl_hbm, out_hbm,
             gi_v, si_v, vl_v, gbuf, gi_t, si_t, vl_t, gbuf_t,
             zbuf, dbuf, pbuf, acc, sem) = refs
        else:
            (x_hbm, gi_hbm, si_hbm, vl_hbm, out_hbm,
             gi_v, si_v, vl_v, gbuf, gi_t, si_t, vl_t, gbuf_t,
             zbuf, dbuf, pbuf, acc, sem) = refs
        c = lax.axis_index("c")
        s = lax.axis_index("s")
        zero16 = jnp.zeros((NLANE,), jnp.float32)

        def zrow(r, carry):
            for j in range(CWV):
                zbuf[r, pl.ds(j * NLANE, NLANE)] = zero16
            return carry
        lax.fori_loop(0, RB, zrow, 0)

        def scale_rows(buf, vals_ref, nrows):
            def srow16(e16, carry):
                base = e16 * NLANE
                v16 = vals_ref[pl.ds(base, NLANE)]
                for l in range(NLANE):
                    vsp = v16[l]
                    row = base + l
                    for j in range(CWV):
                        sl = pl.ds(j * NLANE, NLANE)
                        buf[row, sl] = buf[row, sl] * vsp
                return carry
            lax.fori_loop(0, nrows // NLANE, srow16, 0)

        for phase in range(NCHUNK // NCORE):
            chunk = c * (NCHUNK // NCORE) + phase

            for t in range(NRB):
                pltpu.sync_copy(zbuf, acc.at[pl.ds(s * RPT + t * RB, RB)])
            plsc.subcore_barrier()

            ebase = s * EPT

            def eblock(b, carry):
                off = ebase + b * BE
                pltpu.sync_copy(gi_hbm.at[pl.ds(off, BE)], gi_v)
                pltpu.sync_copy(si_hbm.at[pl.ds(off, BE)], si_v)
                pltpu.sync_copy(vl_hbm.at[pl.ds(off, BE)], vl_v)
                pltpu.async_copy(x_hbm.at[chunk].at[gi_v], gbuf, sem).wait()
                scale_rows(gbuf, vl_v, BE)
                pltpu.sync_copy(gbuf, acc.at[si_v], add=True)
                return carry
            lax.fori_loop(0, NBLK, eblock, 0)

            if TAIL:
                off = ebase + NBLK * BE
                pltpu.sync_copy(gi_hbm.at[pl.ds(off, TAIL)], gi_t)
                pltpu.sync_copy(si_hbm.at[pl.ds(off, TAIL)], si_t)
                pltpu.sync_copy(vl_hbm.at[pl.ds(off, TAIL)], vl_t)
                pltpu.async_copy(x_hbm.at[chunk].at[gi_t], gbuf_t, sem).wait()
                scale_rows(gbuf_t, vl_t, TAIL)
                pltpu.sync_copy(gbuf_t, acc.at[si_t], add=True)
            plsc.subcore_barrier()

            for t in range(NRB):
                row0 = s * RPT + t * RB
                pltpu.sync_copy(acc.at[pl.ds(row0, RB)], dbuf)
                if cheb:
                    pltpu.sync_copy(xp_hbm.at[chunk].at[pl.ds(row0, RB)], pbuf)

                    def crow(r, carry):
                        for j in range(CWV):
                            sl = pl.ds(j * NLANE, NLANE)
                            dbuf[r, sl] = 2.0 * dbuf[r, sl] - pbuf[r, sl]
                        return carry
                    lax.fori_loop(0, RB, crow, 0)
                pltpu.sync_copy(dbuf, out_hbm.at[chunk].at[pl.ds(row0, RB)])
            plsc.subcore_barrier()

    return pl.kernel(
        body,
        out_type=jax.ShapeDtypeStruct((NCHUNK, N_PAD, CW), jnp.float32),
        mesh=mesh,
        scratch_types=scratch,
        compiler_params=pltpu.CompilerParams(use_tc_tiling_on_sc=False),
    )


_spmm_plain = _make_spmm(cheb=False)
_spmm_cheb = _make_spmm(cheb=True)


_NB = 4000
_NGRID = (N_NODES * BATCH) // _NB


def _g1_body(x_ref, hxt_ref, w_ref, b_ref, st2_ref, u_ref):
    y = jnp.dot(x_ref[...], w_ref[...], preferred_element_type=jnp.float32)
    y = jax.nn.sigmoid(y + b_ref[...])
    r = y[:, :UNITS]
    u = y[:, UNITS:]
    st2_ref[...] = r * hxt_ref[...]
    u_ref[...] = u


def _g2_body(x_ref, hxt_ref, u_ref, w_ref, b_ref, out_ref):
    y = jnp.dot(x_ref[...], w_ref[...], preferred_element_type=jnp.float32)
    cand = jnp.tanh(y + b_ref[...])
    u = u_ref[...]
    out_ref[...] = u * hxt_ref[...] + (1.0 - u) * cand


_KIN = FEAT * (2 * CHEB_K + 1)

_g1 = pl.pallas_call(
    _g1_body,
    grid=(_NGRID,),
    in_specs=[
        pl.BlockSpec((_NB, _KIN), lambda i: (i, 0)),
        pl.BlockSpec((_NB, UNITS), lambda i: (i, 0)),
        pl.BlockSpec((_KIN, 2 * UNITS), lambda i: (0, 0)),
        pl.BlockSpec((1, 2 * UNITS), lambda i: (0, 0)),
    ],
    out_specs=[
        pl.BlockSpec((_NB, UNITS), lambda i: (i, 0)),
        pl.BlockSpec((_NB, UNITS), lambda i: (i, 0)),
    ],
    out_shape=[
        jax.ShapeDtypeStruct((N_NODES * BATCH, UNITS), jnp.float32),
        jax.ShapeDtypeStruct((N_NODES * BATCH, UNITS), jnp.float32),
    ],
)

_g2 = pl.pallas_call(
    _g2_body,
    grid=(_NGRID,),
    in_specs=[
        pl.BlockSpec((_NB, _KIN), lambda i: (i, 0)),
        pl.BlockSpec((_NB, UNITS), lambda i: (i, 0)),
        pl.BlockSpec((_NB, UNITS), lambda i: (i, 0)),
        pl.BlockSpec((_KIN, UNITS), lambda i: (0, 0)),
        pl.BlockSpec((1, UNITS), lambda i: (0, 0)),
    ],
    out_specs=pl.BlockSpec((_NB, UNITS), lambda i: (i, 0)),
    out_shape=jax.ShapeDtypeStruct((N_NODES * BATCH, UNITS), jnp.float32),
)


def _pack(state_nbi, inp_t):
    x = jnp.concatenate([inp_t, state_nbi], axis=2)
    x = jnp.pad(x, ((0, 0), (0, 0), (0, FEAT_PAD - FEAT)))
    x = x.reshape(N_NODES, NCHUNK, CW).transpose(1, 0, 2)
    x = jnp.pad(x, ((0, 0), (0, N_PAD - N_NODES), (0, 0)))
    return x


def _unpack(xc):
    x = xc[:, :N_NODES].transpose(1, 0, 2).reshape(N_NODES, BATCH, FEAT_PAD)
    return x[:, :, :FEAT].reshape(N_NODES * BATCH, FEAT)


def _diffuse(x0c, rows, cols, vals):
    y1 = _spmm_plain(x0c, cols, rows, vals)
    y2 = _spmm_cheb(y1, x0c, cols, rows, vals)
    y3 = _spmm_plain(y1, rows, cols, vals)
    y4 = _spmm_cheb(y3, y1, rows, cols, vals)
    return (x0c, y1, y2, y3, y4)


def _xcat(states):
    cols = [_unpack(s) for s in states]
    x = jnp.stack(cols, axis=-1)
    return x.reshape(N_NODES * BATCH, _KIN)


def kernel(inputs, hx, rows, cols, vals, W_ru, b_ru, W_c, b_c):
    inp_t = inputs.reshape(BATCH, N_NODES, IN_DIM).transpose(1, 0, 2)
    hx_t = hx.reshape(BATCH, N_NODES, UNITS).transpose(1, 0, 2)
    hxt_flat = hx_t.reshape(N_NODES * BATCH, UNITS)

    x0c = _pack(hx_t, inp_t)
    xs1 = _diffuse(x0c, rows, cols, vals)
    st2, u = _g1(_xcat(xs1), hxt_flat, W_ru, b_ru.reshape(1, -1))

    x0c2 = _pack(st2.reshape(N_NODES, BATCH, UNITS), inp_t)
    xs2 = _diffuse(x0c2, rows, cols, vals)
    new = _g2(_xcat(xs2), hxt_flat, u, W_c, b_c.reshape(1, -1))

    return new.reshape(N_NODES, BATCH, UNITS).transpose(1, 0, 2).reshape(
        BATCH, N_NODES * UNITS)


# speedup vs baseline: 2.1004x; 1.1149x over previous
import functools

import jax
import jax.numpy as jnp
from jax import lax
from jax.experimental import pallas as pl
from jax.experimental.pallas import tpu as pltpu
from jax.experimental.pallas import tpu_sc as plsc

N_NODES = 10000
N_PAD = 10240
BATCH = 8
IN_DIM = 2
UNITS = 64
CHEB_K = 2
NUM_EDGES = 160000

FEAT = IN_DIM + UNITS
FEAT_PAD = 72
ROW_W = BATCH * FEAT_PAD
NCHUNK = 6
CW = ROW_W // NCHUNK
NSUB = 16
NCORE = 2
BE = 128
NBLK = 80
E_PAD = NSUB * NBLK * BE
EROWS = E_PAD // BE
NRING = 4
RPT = N_PAD // NSUB
RB = 128
NRB = RPT // RB
NLANE = 16
CWV = CW // NLANE


def _make_spmm(cheb: bool):
    mesh = plsc.VectorSubcoreMesh(
        core_axis_name="c", subcore_axis_name="s",
        num_cores=NCORE, num_subcores=NSUB)

    scratch = [
        pltpu.VMEM((NRING, BE), jnp.int32),
        pltpu.VMEM((NRING, BE), jnp.int32),
        pltpu.VMEM((NRING, BE), jnp.float32),
        pltpu.VMEM((BE, CW), jnp.float32),
        pltpu.VMEM((BE, CW), jnp.float32),
        pltpu.VMEM((RB, CW), jnp.float32),
        pltpu.VMEM((RB, CW), jnp.float32),
        pltpu.VMEM_SHARED((N_PAD, CW), jnp.float32),
        [pltpu.SemaphoreType.DMA] * NRING,
        pltpu.SemaphoreType.DMA,
        pltpu.SemaphoreType.DMA,
    ]

    def body(*refs):
        if cheb:
            (x_hbm, xp_hbm, gi_hbm, si_hbm, vl_hbm, out_hbm,
             gi_r, si_r, vl_r, gbuf0, gbuf1,
             dbuf, pbuf, acc, semi, semg0, semg1) = refs
        else:
            (x_hbm, gi_hbm, si_hbm, vl_hbm, out_hbm,
             gi_r, si_r, vl_r, gbuf0, gbuf1,
             dbuf, pbuf, acc, semi, semg0, semg1) = refs
        c = lax.axis_index("c")
        s = lax.axis_index("s")
        zero16 = jnp.zeros((NLANE,), jnp.float32)
        erow0 = s * NBLK
        gbufs = (gbuf0, gbuf1)
        semgs = (semg0, semg1)

        def fire_idx(b, slot):
            row = erow0 + b
            pltpu.async_copy(gi_hbm.at[pl.ds(row, 1)],
                             gi_r.at[pl.ds(slot, 1)], semi[slot])
            pltpu.async_copy(si_hbm.at[pl.ds(row, 1)],
                             si_r.at[pl.ds(slot, 1)], semi[slot])
            pltpu.async_copy(vl_hbm.at[pl.ds(row, 1)],
                             vl_r.at[pl.ds(slot, 1)], semi[slot])

        def wait_idx(b, slot):
            row = erow0 + b
            pltpu.make_async_copy(gi_hbm.at[pl.ds(row, 1)],
                                  gi_r.at[pl.ds(slot, 1)], semi[slot]).wait()
            pltpu.make_async_copy(si_hbm.at[pl.ds(row, 1)],
                                  si_r.at[pl.ds(slot, 1)], semi[slot]).wait()
            pltpu.make_async_copy(vl_hbm.at[pl.ds(row, 1)],
                                  vl_r.at[pl.ds(slot, 1)], semi[slot]).wait()

        def scale_rows(buf, slot):
            def srow16(e16, carry):
                base = e16 * NLANE
                v16 = vl_r[slot, pl.ds(base, NLANE)]
                for l in range(NLANE):
                    vsp = v16[l]
                    row = base + l
                    for j in range(CWV):
                        sl = pl.ds(j * NLANE, NLANE)
                        buf[row, sl] = buf[row, sl] * vsp
                return carry
            lax.fori_loop(0, BE // NLANE, srow16, 0)

        for phase in range(NCHUNK // NCORE):
            chunk = c * (NCHUNK // NCORE) + phase

            def zrow(r, carry):
                for j in range(CWV):
                    dbuf[r, pl.ds(j * NLANE, NLANE)] = zero16
                return carry
            lax.fori_loop(0, RB, zrow, 0)
            for t in range(NRB):
                pltpu.sync_copy(dbuf, acc.at[pl.ds(s * RPT + t * RB, RB)])
            plsc.subcore_barrier()

            xc = x_hbm.at[chunk]
            for p in range(NRING):
                fire_idx(p, p)
            wait_idx(0, 0)
            pltpu.async_copy(xc.at[gi_r.at[0]], gbuf0, semg0)
            wait_idx(1, 1)
            pltpu.async_copy(xc.at[gi_r.at[1]], gbuf1, semg1)

            def quad(q, carry):
                base_b = q * NRING
                for off in range(NRING):
                    b = base_b + off
                    gs = off % 2
                    gbuf, semg = gbufs[gs], semgs[gs]
                    pltpu.make_async_copy(xc.at[gi_r.at[off]], gbuf,
                                          semg).wait()
                    scale_rows(gbuf, off)
                    pltpu.sync_copy(gbuf, acc.at[si_r.at[off]], add=True)

                    @pl.when(b + NRING < NBLK)
                    def _():
                        fire_idx(b + NRING, off)

                    @pl.when(b + 2 < NBLK)
                    def _():
                        nslot = (off + 2) % NRING
                        wait_idx(b + 2, nslot)
                        pltpu.async_copy(xc.at[gi_r.at[nslot]], gbuf, semg)
                return carry
            lax.fori_loop(0, NBLK // NRING, quad, 0)

            plsc.subcore_barrier()

            for t in range(NRB):
                row0 = s * RPT + t * RB
                pltpu.sync_copy(acc.at[pl.ds(row0, RB)], dbuf)
                if cheb:
                    pltpu.sync_copy(xp_hbm.at[chunk].at[pl.ds(row0, RB)], pbuf)

                    def crow(r, carry):
                        for j in range(CWV):
                            sl = pl.ds(j * NLANE, NLANE)
                            dbuf[r, sl] = 2.0 * dbuf[r, sl] - pbuf[r, sl]
                        return carry
                    lax.fori_loop(0, RB, crow, 0)
                pltpu.sync_copy(dbuf, out_hbm.at[chunk].at[pl.ds(row0, RB)])
            plsc.subcore_barrier()

    return pl.kernel(
        body,
        out_type=jax.ShapeDtypeStruct((NCHUNK, N_PAD, CW), jnp.float32),
        mesh=mesh,
        scratch_types=scratch,
        compiler_params=pltpu.CompilerParams(use_tc_tiling_on_sc=False),
    )


_spmm_plain = _make_spmm(cheb=False)
_spmm_cheb = _make_spmm(cheb=True)


_NB = 4000
_NGRID = (N_NODES * BATCH) // _NB


def _g1_body(x_ref, hxt_ref, w_ref, b_ref, st2_ref, u_ref):
    y = jnp.dot(x_ref[...], w_ref[...], preferred_element_type=jnp.float32)
    y = jax.nn.sigmoid(y + b_ref[...])
    r = y[:, :UNITS]
    u = y[:, UNITS:]
    st2_ref[...] = r * hxt_ref[...]
    u_ref[...] = u


def _g2_body(x_ref, hxt_ref, u_ref, w_ref, b_ref, out_ref):
    y = jnp.dot(x_ref[...], w_ref[...], preferred_element_type=jnp.float32)
    cand = jnp.tanh(y + b_ref[...])
    u = u_ref[...]
    out_ref[...] = u * hxt_ref[...] + (1.0 - u) * cand


_KIN = FEAT * (2 * CHEB_K + 1)

_g1 = pl.pallas_call(
    _g1_body,
    grid=(_NGRID,),
    in_specs=[
        pl.BlockSpec((_NB, _KIN), lambda i: (i, 0)),
        pl.BlockSpec((_NB, UNITS), lambda i: (i, 0)),
        pl.BlockSpec((_KIN, 2 * UNITS), lambda i: (0, 0)),
        pl.BlockSpec((1, 2 * UNITS), lambda i: (0, 0)),
    ],
    out_specs=[
        pl.BlockSpec((_NB, UNITS), lambda i: (i, 0)),
        pl.BlockSpec((_NB, UNITS), lambda i: (i, 0)),
    ],
    out_shape=[
        jax.ShapeDtypeStruct((N_NODES * BATCH, UNITS), jnp.float32),
        jax.ShapeDtypeStruct((N_NODES * BATCH, UNITS), jnp.float32),
    ],
)

_g2 = pl.pallas_call(
    _g2_body,
    grid=(_NGRID,),
    in_specs=[
        pl.BlockSpec((_NB, _KIN), lambda i: (i, 0)),
        pl.BlockSpec((_NB, UNITS), lambda i: (i, 0)),
        pl.BlockSpec((_NB, UNITS), lambda i: (i, 0)),
        pl.BlockSpec((_KIN, UNITS), lambda i: (0, 0)),
        pl.BlockSpec((1, UNITS), lambda i: (0, 0)),
    ],
    out_specs=pl.BlockSpec((_NB, UNITS), lambda i: (i, 0)),
    out_shape=jax.ShapeDtypeStruct((N_NODES * BATCH, UNITS), jnp.float32),
)


def _pack(state_nbi, inp_t):
    x = jnp.concatenate([inp_t, state_nbi], axis=2)
    x = jnp.pad(x, ((0, 0), (0, 0), (0, FEAT_PAD - FEAT)))
    x = x.reshape(N_NODES, NCHUNK, CW).transpose(1, 0, 2)
    x = jnp.pad(x, ((0, 0), (0, N_PAD - N_NODES), (0, 0)))
    return x


def _unpack(xc):
    x = xc[:, :N_NODES].transpose(1, 0, 2).reshape(N_NODES, BATCH, FEAT_PAD)
    return x[:, :, :FEAT].reshape(N_NODES * BATCH, FEAT)


def _pad_edges(a):
    return jnp.pad(a, (0, E_PAD - NUM_EDGES)).reshape(EROWS, BE)


def _diffuse(x0c, rows2, cols2, vals2):
    y1 = _spmm_plain(x0c, cols2, rows2, vals2)
    y2 = _spmm_cheb(y1, x0c, cols2, rows2, vals2)
    y3 = _spmm_plain(y1, rows2, cols2, vals2)
    y4 = _spmm_cheb(y3, y1, rows2, cols2, vals2)
    return (x0c, y1, y2, y3, y4)


def _xcat(states):
    cols = [_unpack(s) for s in states]
    x = jnp.stack(cols, axis=-1)
    return x.reshape(N_NODES * BATCH, _KIN)


def kernel(inputs, hx, rows, cols, vals, W_ru, b_ru, W_c, b_c):
    inp_t = inputs.reshape(BATCH, N_NODES, IN_DIM).transpose(1, 0, 2)
    hx_t = hx.reshape(BATCH, N_NODES, UNITS).transpose(1, 0, 2)
    hxt_flat = hx_t.reshape(N_NODES * BATCH, UNITS)

    rows2, cols2, vals2 = _pad_edges(rows), _pad_edges(cols), _pad_edges(vals)

    x0c = _pack(hx_t, inp_t)
    xs1 = _diffuse(x0c, rows2, cols2, vals2)
    st2, u = _g1(_xcat(xs1), hxt_flat, W_ru, b_ru.reshape(1, -1))

    x0c2 = _pack(st2.reshape(N_NODES, BATCH, UNITS), inp_t)
    xs2 = _diffuse(x0c2, rows2, cols2, vals2)
    new = _g2(_xcat(xs2), hxt_flat, u, W_c, b_c.reshape(1, -1))

    return new.reshape(N_NODES, BATCH, UNITS).transpose(1, 0, 2).reshape(
        BATCH, N_NODES * UNITS)


# speedup vs baseline: 2.1168x; 1.0078x over previous
import functools

import jax
import jax.numpy as jnp
from jax import lax
from jax.experimental import pallas as pl
from jax.experimental.pallas import tpu as pltpu
from jax.experimental.pallas import tpu_sc as plsc

N_NODES = 10000
N_PAD = 10240
BATCH = 8
IN_DIM = 2
UNITS = 64
CHEB_K = 2
NUM_EDGES = 160000

FEAT = IN_DIM + UNITS
FEAT_PAD = 72
ROW_W = BATCH * FEAT_PAD
NCHUNK = 6
CW = ROW_W // NCHUNK
NSUB = 16
NCORE = 2
BE = 128
NBLK = 80
E_PAD = NSUB * NBLK * BE
EROWS = E_PAD // BE
NRING = 8
NGB = 4
RPT = N_PAD // NSUB
RB = 128
NRB = RPT // RB
NLANE = 16
CWV = CW // NLANE


def _make_spmm(cheb: bool):
    mesh = plsc.VectorSubcoreMesh(
        core_axis_name="c", subcore_axis_name="s",
        num_cores=NCORE, num_subcores=NSUB)

    scratch = [
        pltpu.VMEM((NRING, BE), jnp.int32),
        pltpu.VMEM((NRING, BE), jnp.int32),
        pltpu.VMEM((NRING, BE), jnp.float32),
        [pltpu.VMEM((BE, CW), jnp.float32)] * NGB,
        pltpu.VMEM_SHARED((N_PAD, CW), jnp.float32),
        [pltpu.SemaphoreType.DMA] * NRING,
        [pltpu.SemaphoreType.DMA] * NGB,
        [pltpu.SemaphoreType.DMA] * NGB,
    ]

    def body(*refs):
        if cheb:
            (x_hbm, xp_hbm, gi_hbm, si_hbm, vl_hbm, out_hbm,
             gi_r, si_r, vl_r, gbufs,
             acc, semi, semg, sems) = refs
        else:
            (x_hbm, gi_hbm, si_hbm, vl_hbm, out_hbm,
             gi_r, si_r, vl_r, gbufs,
             acc, semi, semg, sems) = refs
        dbuf = gbufs[0]
        pbuf = gbufs[1]
        c = lax.axis_index("c")
        s = lax.axis_index("s")
        zero16 = jnp.zeros((NLANE,), jnp.float32)
        erow0 = s * NBLK

        def fire_idx(b, slot):
            row = erow0 + b
            pltpu.async_copy(gi_hbm.at[pl.ds(row, 1)],
                             gi_r.at[pl.ds(slot, 1)], semi[slot])
            pltpu.async_copy(si_hbm.at[pl.ds(row, 1)],
                             si_r.at[pl.ds(slot, 1)], semi[slot])
            pltpu.async_copy(vl_hbm.at[pl.ds(row, 1)],
                             vl_r.at[pl.ds(slot, 1)], semi[slot])

        def wait_idx(b, slot):
            row = erow0 + b
            pltpu.make_async_copy(gi_hbm.at[pl.ds(row, 1)],
                                  gi_r.at[pl.ds(slot, 1)], semi[slot]).wait()
            pltpu.make_async_copy(si_hbm.at[pl.ds(row, 1)],
                                  si_r.at[pl.ds(slot, 1)], semi[slot]).wait()
            pltpu.make_async_copy(vl_hbm.at[pl.ds(row, 1)],
                                  vl_r.at[pl.ds(slot, 1)], semi[slot]).wait()

        def scale_rows(buf, slot):
            def srow16(e16, carry):
                base = e16 * NLANE
                v16 = vl_r[slot, pl.ds(base, NLANE)]
                for l in range(NLANE):
                    vsp = v16[l]
                    row = base + l
                    for j in range(CWV):
                        sl = pl.ds(j * NLANE, NLANE)
                        buf[row, sl] = buf[row, sl] * vsp
                return carry
            lax.fori_loop(0, BE // NLANE, srow16, 0)

        for phase in range(NCHUNK // NCORE):
            chunk = c * (NCHUNK // NCORE) + phase

            def zrow(r, carry):
                for j in range(CWV):
                    dbuf[r, pl.ds(j * NLANE, NLANE)] = zero16
                return carry
            lax.fori_loop(0, RB, zrow, 0)
            for t in range(NRB):
                pltpu.sync_copy(dbuf, acc.at[pl.ds(s * RPT + t * RB, RB)])
            plsc.subcore_barrier()

            xc = x_hbm.at[chunk]
            for p in range(6):
                fire_idx(p, p)
            for p in range(2):
                wait_idx(p, p)
                pltpu.async_copy(xc.at[gi_r.at[p]], gbufs[p], semg[p])

            def octet(q, carry):
                base_b = q * NRING
                for off in range(NRING):
                    b = base_b + off
                    gs = off % NGB
                    gbuf = gbufs[gs]
                    pltpu.make_async_copy(xc.at[gi_r.at[off]], gbuf,
                                          semg[gs]).wait()
                    scale_rows(gbuf, off)
                    pltpu.async_copy(gbuf, acc.at[si_r.at[off]], sems[gs],
                                     add=True)
                    pgs = (off - 2) % NGB

                    @pl.when(b >= 2)
                    def _():
                        pltpu.make_async_copy(
                            gbufs[pgs], acc.at[si_r.at[(off - 2) % NRING]],
                            sems[pgs]).wait()

                    @pl.when(b + 6 < NBLK)
                    def _():
                        fire_idx(b + 6, (off - 2) % NRING)

                    @pl.when(b + 2 < NBLK)
                    def _():
                        nslot = (off + 2) % NRING
                        ngs = (off + 2) % NGB
                        wait_idx(b + 2, nslot)
                        pltpu.async_copy(xc.at[gi_r.at[nslot]], gbufs[ngs],
                                         semg[ngs])
                return carry
            lax.fori_loop(0, NBLK // NRING, octet, 0)

            for b in (NBLK - 2, NBLK - 1):
                pltpu.make_async_copy(
                    gbufs[b % NGB], acc.at[si_r.at[b % NRING]],
                    sems[b % NGB]).wait()

            plsc.subcore_barrier()

            for t in range(NRB):
                row0 = s * RPT + t * RB
                pltpu.sync_copy(acc.at[pl.ds(row0, RB)], dbuf)
                if cheb:
                    pltpu.sync_copy(xp_hbm.at[chunk].at[pl.ds(row0, RB)], pbuf)

                    def crow(r, carry):
                        for j in range(CWV):
                            sl = pl.ds(j * NLANE, NLANE)
                            dbuf[r, sl] = 2.0 * dbuf[r, sl] - pbuf[r, sl]
                        return carry
                    lax.fori_loop(0, RB, crow, 0)
                pltpu.sync_copy(dbuf, out_hbm.at[chunk].at[pl.ds(row0, RB)])
            plsc.subcore_barrier()

    return pl.kernel(
        body,
        out_type=jax.ShapeDtypeStruct((NCHUNK, N_PAD, CW), jnp.float32),
        mesh=mesh,
        scratch_types=scratch,
        compiler_params=pltpu.CompilerParams(use_tc_tiling_on_sc=False),
    )


_spmm_plain = _make_spmm(cheb=False)
_spmm_cheb = _make_spmm(cheb=True)


_NB = 4000
_NGRID = (N_NODES * BATCH) // _NB


def _g1_body(x_ref, hxt_ref, w_ref, b_ref, st2_ref, u_ref):
    y = jnp.dot(x_ref[...], w_ref[...], preferred_element_type=jnp.float32)
    y = jax.nn.sigmoid(y + b_ref[...])
    r = y[:, :UNITS]
    u = y[:, UNITS:]
    st2_ref[...] = r * hxt_ref[...]
    u_ref[...] = u


def _g2_body(x_ref, hxt_ref, u_ref, w_ref, b_ref, out_ref):
    y = jnp.dot(x_ref[...], w_ref[...], preferred_element_type=jnp.float32)
    cand = jnp.tanh(y + b_ref[...])
    u = u_ref[...]
    out_ref[...] = u * hxt_ref[...] + (1.0 - u) * cand


_KIN = FEAT * (2 * CHEB_K + 1)

_g1 = pl.pallas_call(
    _g1_body,
    grid=(_NGRID,),
    in_specs=[
        pl.BlockSpec((_NB, _KIN), lambda i: (i, 0)),
        pl.BlockSpec((_NB, UNITS), lambda i: (i, 0)),
        pl.BlockSpec((_KIN, 2 * UNITS), lambda i: (0, 0)),
        pl.BlockSpec((1, 2 * UNITS), lambda i: (0, 0)),
    ],
    out_specs=[
        pl.BlockSpec((_NB, UNITS), lambda i: (i, 0)),
        pl.BlockSpec((_NB, UNITS), lambda i: (i, 0)),
    ],
    out_shape=[
        jax.ShapeDtypeStruct((N_NODES * BATCH, UNITS), jnp.float32),
        jax.ShapeDtypeStruct((N_NODES * BATCH, UNITS), jnp.float32),
    ],
)

_g2 = pl.pallas_call(
    _g2_body,
    grid=(_NGRID,),
    in_specs=[
        pl.BlockSpec((_NB, _KIN), lambda i: (i, 0)),
        pl.BlockSpec((_NB, UNITS), lambda i: (i, 0)),
        pl.BlockSpec((_NB, UNITS), lambda i: (i, 0)),
        pl.BlockSpec((_KIN, UNITS), lambda i: (0, 0)),
        pl.BlockSpec((1, UNITS), lambda i: (0, 0)),
    ],
    out_specs=pl.BlockSpec((_NB, UNITS), lambda i: (i, 0)),
    out_shape=jax.ShapeDtypeStruct((N_NODES * BATCH, UNITS), jnp.float32),
)


def _pack(state_nbi, inp_t):
    x = jnp.concatenate([inp_t, state_nbi], axis=2)
    x = jnp.pad(x, ((0, 0), (0, 0), (0, FEAT_PAD - FEAT)))
    x = x.reshape(N_NODES, NCHUNK, CW).transpose(1, 0, 2)
    x = jnp.pad(x, ((0, 0), (0, N_PAD - N_NODES), (0, 0)))
    return x


def _unpack(xc):
    x = xc[:, :N_NODES].transpose(1, 0, 2).reshape(N_NODES, BATCH, FEAT_PAD)
    return x[:, :, :FEAT].reshape(N_NODES * BATCH, FEAT)


def _pad_edges(a):
    return jnp.pad(a, (0, E_PAD - NUM_EDGES)).reshape(EROWS, BE)


def _diffuse(x0c, rows2, cols2, vals2):
    y1 = _spmm_plain(x0c, cols2, rows2, vals2)
    y2 = _spmm_cheb(y1, x0c, cols2, rows2, vals2)
    y3 = _spmm_plain(y1, rows2, cols2, vals2)
    y4 = _spmm_cheb(y3, y1, rows2, cols2, vals2)
    return (x0c, y1, y2, y3, y4)


def _xcat(states):
    cols = [_unpack(s) for s in states]
    x = jnp.stack(cols, axis=-1)
    return x.reshape(N_NODES * BATCH, _KIN)


def kernel(inputs, hx, rows, cols, vals, W_ru, b_ru, W_c, b_c):
    inp_t = inputs.reshape(BATCH, N_NODES, IN_DIM).transpose(1, 0, 2)
    hx_t = hx.reshape(BATCH, N_NODES, UNITS).transpose(1, 0, 2)
    hxt_flat = hx_t.reshape(N_NODES * BATCH, UNITS)

    rows2, cols2, vals2 = _pad_edges(rows), _pad_edges(cols), _pad_edges(vals)

    x0c = _pack(hx_t, inp_t)
    xs1 = _diffuse(x0c, rows2, cols2, vals2)
    st2, u = _g1(_xcat(xs1), hxt_flat, W_ru, b_ru.reshape(1, -1))

    x0c2 = _pack(st2.reshape(N_NODES, BATCH, UNITS), inp_t)
    xs2 = _diffuse(x0c2, rows2, cols2, vals2)
    new = _g2(_xcat(xs2), hxt_flat, u, W_c, b_c.reshape(1, -1))

    return new.reshape(N_NODES, BATCH, UNITS).transpose(1, 0, 2).reshape(
        BATCH, N_NODES * UNITS)


# speedup vs baseline: 2.7213x; 1.2856x over previous
import functools

import jax
import jax.numpy as jnp
from jax import lax
from jax.experimental import pallas as pl
from jax.experimental.pallas import tpu as pltpu
from jax.experimental.pallas import tpu_sc as plsc

N_NODES = 10000
N_PAD = 10240
BATCH = 8
IN_DIM = 2
UNITS = 64
CHEB_K = 2
NUM_EDGES = 160000

FEAT = IN_DIM + UNITS
FEAT_PAD = 72
ROW_W = BATCH * FEAT_PAD
NCHUNK = 6
CW = ROW_W // NCHUNK
NSUB = 16
NCORE = 2
BE = 128
NBLK = 80
E_PAD = NSUB * NBLK * BE
EROWS = E_PAD // BE
NRING = 8
NGB = 4
RPT = N_PAD // NSUB
RB = 128
NRB = RPT // RB
NLANE = 16
CWV = CW // NLANE


def _make_spmm(cheb: bool):
    mesh = plsc.VectorSubcoreMesh(
        core_axis_name="c", subcore_axis_name="s",
        num_cores=NCORE, num_subcores=NSUB)

    scratch = [
        pltpu.VMEM((NRING, BE), jnp.int32),
        pltpu.VMEM((NRING, BE), jnp.int32),
        pltpu.VMEM((NRING, BE), jnp.float32),
        [pltpu.VMEM((BE, CW), jnp.float32)] * NGB,
        pltpu.VMEM_SHARED((N_PAD, CW), jnp.float32),
        [pltpu.SemaphoreType.DMA] * NRING,
        [pltpu.SemaphoreType.DMA] * NGB,
        [pltpu.SemaphoreType.DMA] * NGB,
    ]

    def body(*refs):
        if cheb:
            (x_hbm, xp_hbm, gi_hbm, si_hbm, vl_hbm, out_hbm, outf_hbm,
             gi_r, si_r, vl_r, gbufs,
             acc, semi, semg, sems) = refs
        else:
            (x_hbm, gi_hbm, si_hbm, vl_hbm, out_hbm, outf_hbm,
             gi_r, si_r, vl_r, gbufs,
             acc, semi, semg, sems) = refs
        dbuf = gbufs[0]
        pbuf = gbufs[1]
        c = lax.axis_index("c")
        s = lax.axis_index("s")
        zero16 = jnp.zeros((NLANE,), jnp.float32)
        erow0 = s * NBLK

        def fire_idx(b, slot):
            row = erow0 + b
            pltpu.async_copy(gi_hbm.at[pl.ds(row, 1)],
                             gi_r.at[pl.ds(slot, 1)], semi[slot])
            pltpu.async_copy(si_hbm.at[pl.ds(row, 1)],
                             si_r.at[pl.ds(slot, 1)], semi[slot])
            pltpu.async_copy(vl_hbm.at[pl.ds(row, 1)],
                             vl_r.at[pl.ds(slot, 1)], semi[slot])

        def wait_idx(b, slot):
            row = erow0 + b
            pltpu.make_async_copy(gi_hbm.at[pl.ds(row, 1)],
                                  gi_r.at[pl.ds(slot, 1)], semi[slot]).wait()
            pltpu.make_async_copy(si_hbm.at[pl.ds(row, 1)],
                                  si_r.at[pl.ds(slot, 1)], semi[slot]).wait()
            pltpu.make_async_copy(vl_hbm.at[pl.ds(row, 1)],
                                  vl_r.at[pl.ds(slot, 1)], semi[slot]).wait()

        def scale_rows(buf, slot):
            def srow16(e16, carry):
                base = e16 * NLANE
                v16 = vl_r[slot, pl.ds(base, NLANE)]
                for l in range(NLANE):
                    vsp = v16[l]
                    row = base + l
                    for j in range(CWV):
                        sl = pl.ds(j * NLANE, NLANE)
                        buf[row, sl] = buf[row, sl] * vsp
                return carry
            lax.fori_loop(0, BE // NLANE, srow16, 0)

        for phase in range(NCHUNK // NCORE):
            chunk = c * (NCHUNK // NCORE) + phase

            def zrow(r, carry):
                for j in range(CWV):
                    dbuf[r, pl.ds(j * NLANE, NLANE)] = zero16
                return carry
            lax.fori_loop(0, RB, zrow, 0)
            for t in range(NRB):
                pltpu.sync_copy(dbuf, acc.at[pl.ds(s * RPT + t * RB, RB)])
            plsc.subcore_barrier()

            xc = x_hbm.at[chunk]
            for p in range(6):
                fire_idx(p, p)
            for p in range(2):
                wait_idx(p, p)
                pltpu.async_copy(xc.at[gi_r.at[p]], gbufs[p], semg[p])

            def octet(q, carry):
                base_b = q * NRING
                for off in range(NRING):
                    b = base_b + off
                    gs = off % NGB
                    gbuf = gbufs[gs]
                    pltpu.make_async_copy(xc.at[gi_r.at[off]], gbuf,
                                          semg[gs]).wait()
                    scale_rows(gbuf, off)
                    pltpu.async_copy(gbuf, acc.at[si_r.at[off]], sems[gs],
                                     add=True)
                    pgs = (off - 2) % NGB

                    @pl.when(b >= 2)
                    def _():
                        pltpu.make_async_copy(
                            gbufs[pgs], acc.at[si_r.at[(off - 2) % NRING]],
                            sems[pgs]).wait()

                    @pl.when(b + 6 < NBLK)
                    def _():
                        fire_idx(b + 6, (off - 2) % NRING)

                    @pl.when(b + 2 < NBLK)
                    def _():
                        nslot = (off + 2) % NRING
                        ngs = (off + 2) % NGB
                        wait_idx(b + 2, nslot)
                        pltpu.async_copy(xc.at[gi_r.at[nslot]], gbufs[ngs],
                                         semg[ngs])
                return carry
            lax.fori_loop(0, NBLK // NRING, octet, 0)

            for b in (NBLK - 2, NBLK - 1):
                pltpu.make_async_copy(
                    gbufs[b % NGB], acc.at[si_r.at[b % NRING]],
                    sems[b % NGB]).wait()

            plsc.subcore_barrier()

            for t in range(NRB):
                row0 = s * RPT + t * RB
                pltpu.sync_copy(acc.at[pl.ds(row0, RB)], dbuf)
                if cheb:
                    pltpu.sync_copy(xp_hbm.at[chunk].at[pl.ds(row0, RB)], pbuf)

                    def crow(r, carry):
                        for j in range(CWV):
                            sl = pl.ds(j * NLANE, NLANE)
                            dbuf[r, sl] = 2.0 * dbuf[r, sl] - pbuf[r, sl]
                        return carry
                    lax.fori_loop(0, RB, crow, 0)
                pltpu.sync_copy(dbuf, out_hbm.at[chunk].at[pl.ds(row0, RB)])
                pltpu.sync_copy(
                    dbuf,
                    outf_hbm.at[pl.ds(row0, RB), pl.ds(chunk * CW, CW)])
            plsc.subcore_barrier()

    return pl.kernel(
        body,
        out_type=[
            jax.ShapeDtypeStruct((NCHUNK, N_PAD, CW), jnp.float32),
            jax.ShapeDtypeStruct((N_PAD, ROW_W), jnp.float32),
        ],
        mesh=mesh,
        scratch_types=scratch,
        compiler_params=pltpu.CompilerParams(use_tc_tiling_on_sc=False),
    )


_spmm_plain = _make_spmm(cheb=False)
_spmm_cheb = _make_spmm(cheb=True)


_NROWS = N_PAD * BATCH
_NB = 4096
_NGRID = _NROWS // _NB
_NMAT = 2 * CHEB_K + 1


def _g1_body(x0_ref, x1_ref, x2_ref, x3_ref, x4_ref, w_ref, b_ref,
             st2_ref, u_ref):
    xs = (x0_ref, x1_ref, x2_ref, x3_ref, x4_ref)
    y = b_ref[...]
    for m in range(_NMAT):
        y = y + jnp.dot(xs[m][...], w_ref[m],
                        preferred_element_type=jnp.float32)
    y = jax.nn.sigmoid(y)
    r = y[:, :UNITS]
    u = y[:, UNITS:]
    st2_ref[...] = r * x0_ref[:, IN_DIM:FEAT]
    u_ref[...] = u


def _g2_body(x0_ref, x1_ref, x2_ref, x3_ref, x4_ref, hx0_ref, u_ref,
             w_ref, b_ref, out_ref):
    xs = (x0_ref, x1_ref, x2_ref, x3_ref, x4_ref)
    y = b_ref[...]
    for m in range(_NMAT):
        y = y + jnp.dot(xs[m][...], w_ref[m],
                        preferred_element_type=jnp.float32)
    cand = jnp.tanh(y)
    u = u_ref[...]
    out_ref[...] = u * hx0_ref[:, IN_DIM:FEAT] + (1.0 - u) * cand


def _xspec():
    return pl.BlockSpec((_NB, FEAT_PAD), lambda i: (i, 0))


_g1 = pl.pallas_call(
    _g1_body,
    grid=(_NGRID,),
    in_specs=[_xspec() for _ in range(_NMAT)] + [
        pl.BlockSpec((_NMAT, FEAT_PAD, 2 * UNITS), lambda i: (0, 0, 0)),
        pl.BlockSpec((1, 2 * UNITS), lambda i: (0, 0)),
    ],
    out_specs=[
        pl.BlockSpec((_NB, UNITS), lambda i: (i, 0)),
        pl.BlockSpec((_NB, UNITS), lambda i: (i, 0)),
    ],
    out_shape=[
        jax.ShapeDtypeStruct((_NROWS, UNITS), jnp.float32),
        jax.ShapeDtypeStruct((_NROWS, UNITS), jnp.float32),
    ],
)

_g2 = pl.pallas_call(
    _g2_body,
    grid=(_NGRID,),
    in_specs=[_xspec() for _ in range(_NMAT)] + [
        _xspec(),
        pl.BlockSpec((_NB, UNITS), lambda i: (i, 0)),
        pl.BlockSpec((_NMAT, FEAT_PAD, UNITS), lambda i: (0, 0, 0)),
        pl.BlockSpec((1, UNITS), lambda i: (0, 0)),
    ],
    out_specs=pl.BlockSpec((_NB, UNITS), lambda i: (i, 0)),
    out_shape=jax.ShapeDtypeStruct((_NROWS, UNITS), jnp.float32),
)


def _pad_edges(a):
    return jnp.pad(a, (0, E_PAD - NUM_EDGES)).reshape(EROWS, BE)


def _pack(flat):
    return flat.reshape(N_PAD, NCHUNK, CW).transpose(1, 0, 2)


def _diffuse(x0c, x0f, rows2, cols2, vals2):
    y1c, y1f = _spmm_plain(x0c, cols2, rows2, vals2)
    _, y2f = _spmm_cheb(y1c, x0c, cols2, rows2, vals2)
    y3c, y3f = _spmm_plain(y1c, rows2, cols2, vals2)
    _, y4f = _spmm_cheb(y3c, y1c, rows2, cols2, vals2)
    return (x0f, y1f, y2f, y3f, y4f)


def _flat_rows(f):
    return f.reshape(_NROWS, FEAT_PAD)


def _split_w(W, out_dim):
    w = W.reshape(FEAT, _NMAT, out_dim).transpose(1, 0, 2)
    return jnp.pad(w, ((0, 0), (0, FEAT_PAD - FEAT), (0, 0)))


def kernel(inputs, hx, rows, cols, vals, W_ru, b_ru, W_c, b_c):
    inp_t = inputs.reshape(BATCH, N_NODES, IN_DIM).transpose(1, 0, 2)
    hx_t = hx.reshape(BATCH, N_NODES, UNITS).transpose(1, 0, 2)

    rows2, cols2, vals2 = _pad_edges(rows), _pad_edges(cols), _pad_edges(vals)

    x0 = jnp.concatenate(
        [inp_t, hx_t,
         jnp.zeros((N_NODES, BATCH, FEAT_PAD - FEAT), jnp.float32)], axis=2)
    x0f = jnp.pad(x0.reshape(N_NODES, ROW_W), ((0, N_PAD - N_NODES), (0, 0)))
    x0c = _pack(x0f)

    xs1 = _diffuse(x0c, x0f, rows2, cols2, vals2)
    st2, u = _g1(*[_flat_rows(f) for f in xs1],
                 _split_w(W_ru, 2 * UNITS), b_ru.reshape(1, -1))

    x0p = jnp.concatenate(
        [jnp.pad(inp_t, ((0, N_PAD - N_NODES), (0, 0), (0, 0))),
         st2.reshape(N_PAD, BATCH, UNITS),
         jnp.zeros((N_PAD, BATCH, FEAT_PAD - FEAT), jnp.float32)], axis=2)
    x0pf = x0p.reshape(N_PAD, ROW_W)
    x0pc = _pack(x0pf)

    xs2 = _diffuse(x0pc, x0pf, rows2, cols2, vals2)
    new = _g2(*[_flat_rows(f) for f in xs2],
              _flat_rows(x0f), u,
              _split_w(W_c, UNITS), b_c.reshape(1, -1))

    return (new.reshape(N_PAD, BATCH, UNITS)[:N_NODES]
            .transpose(1, 0, 2).reshape(BATCH, N_NODES * UNITS))
